# baseline (speedup 1.0000x reference)
import jax
import jax.numpy as jnp
from jax.experimental import pallas as pl
from jax.experimental.pallas import tpu as pltpu

NUM_VECTOR = 8
LAYER_IDX = 16
T = 8


def _body(x_ref, w_ref, b_ref, pool_ref, o_ref):
    tail = x_ref[:, :, :]
    act = tail[:, T - 1, :]
    scores = jax.lax.dot_general(
        act, w_ref[...], (((1,), (0,)), ((), ())),
        precision=jax.lax.Precision.HIGHEST)
    scores = scores + b_ref[...]
    idx = jnp.argmax(scores, axis=1).reshape(-1, 1)
    keep = idx == NUM_VECTOR
    onehot = (jax.lax.broadcasted_iota(jnp.int32, (idx.shape[0], NUM_VECTOR), 1)
              == idx).astype(jnp.float32)
    repl = jax.lax.dot_general(
        onehot, pool_ref[...], (((1,), (0,)), ((), ())),
        precision=jax.lax.Precision.HIGHEST)
    new_last = jnp.where(keep, act, repl)
    o_ref[:, :, :] = jnp.concatenate(
        [tail[:, : T - 1, :], new_last[:, None, :]], axis=1)


def kernel(x, vector_pool, gate_W, gate_b):
    B, S, H = x.shape
    pool_layer = vector_pool[:, LAYER_IDX, :]
    gate_b2 = gate_b.reshape(1, -1)
    return pl.pallas_call(
        _body,
        grid=(1,),
        in_specs=[
            pl.BlockSpec((B, T, H), lambda i: (0, S // T - 1, 0)),
            pl.BlockSpec((H, NUM_VECTOR + 1), lambda i: (0, 0)),
            pl.BlockSpec((1, NUM_VECTOR + 1), lambda i: (0, 0)),
            pl.BlockSpec((NUM_VECTOR, H), lambda i: (0, 0)),
        ],
        out_specs=pl.BlockSpec((B, T, H), lambda i: (0, S // T - 1, 0)),
        out_shape=jax.ShapeDtypeStruct((B, S, H), x.dtype),
        input_output_aliases={0: 0},
    )(x, gate_W, gate_b2, pool_layer)
